# baseline (speedup 1.0000x reference)
import functools

import jax
import jax.numpy as jnp
from jax import lax
from jax.experimental import pallas as pl
from jax.experimental.pallas import tpu as pltpu
from jax.experimental.pallas import tpu_sc as plsc

_B, _L, _D = 4, 4096, 1024
_NC, _NS = 2, 16
_NW = _NC * _NS
_ROWS_PER_W = _L // _NW
_CHUNK = 32
_NCH = _ROWS_PER_W // _CHUNK


def _make_pe_kernel():
    mesh = plsc.VectorSubcoreMesh(core_axis_name="c", subcore_axis_name="s")

    @functools.partial(
        pl.kernel,
        out_type=jax.ShapeDtypeStruct((_B, _L, _D), jnp.float32),
        mesh=mesh,
        scratch_types=[
            pltpu.VMEM((_CHUNK, _D), jnp.float32),
            pltpu.VMEM((_CHUNK, _D), jnp.float32),
            pltpu.SemaphoreType.DMA,
            pltpu.SemaphoreType.DMA,
            pltpu.SemaphoreType.DMA,
            pltpu.SemaphoreType.DMA,
        ],
    )
    def pe_kernel(table_hbm, out_hbm, buf0, buf1, rsem0, rsem1, wsem0, wsem1):
        wid = lax.axis_index("s") * _NC + lax.axis_index("c")
        base = wid * _ROWS_PER_W
        bufs = (buf0, buf1)
        rsems = (rsem0, rsem1)
        wsems = (wsem0, wsem1)

        def start_read(c):
            return pltpu.async_copy(
                table_hbm.at[pl.ds(base + c * _CHUNK, _CHUNK)],
                bufs[c % 2], rsems[c % 2])

        def start_writes(c):
            return [
                pltpu.async_copy(
                    bufs[c % 2],
                    out_hbm.at[b, pl.ds(base + c * _CHUNK, _CHUNK)],
                    wsems[c % 2])
                for b in range(_B)
            ]

        reads = [None] * _NCH
        writes = [None] * _NCH
        reads[0] = start_read(0)
        reads[1] = start_read(1)
        for c in range(_NCH):
            reads[c].wait()
            writes[c] = start_writes(c)
            nxt = c + 2
            if nxt < _NCH:
                for w in writes[c]:
                    w.wait()
                reads[nxt] = start_read(nxt)
        for w in writes[_NCH - 2]:
            w.wait()
        for w in writes[_NCH - 1]:
            w.wait()

    return pe_kernel


_pe = _make_pe_kernel()


def kernel(seq_h, pos_table):
    del seq_h
    return _pe(pos_table)


# speedup vs baseline: 1.0160x; 1.0160x over previous
import functools

import jax
import jax.numpy as jnp
from jax import lax
from jax.experimental import pallas as pl
from jax.experimental.pallas import tpu as pltpu
from jax.experimental.pallas import tpu_sc as plsc

_B, _L, _D = 4, 4096, 1024
_NC, _NS = 2, 16
_NW = _NC * _NS
_ROWS_PER_W = _L // _NW
_CHUNK = 64


def _make_pe_kernel():
    mesh = plsc.VectorSubcoreMesh(core_axis_name="c", subcore_axis_name="s")

    @functools.partial(
        pl.kernel,
        out_type=jax.ShapeDtypeStruct((_B, _L, _D), jnp.float32),
        mesh=mesh,
        scratch_types=[
            pltpu.VMEM((_CHUNK, _D), jnp.float32),
        ],
    )
    def pe_kernel(table_hbm, out_hbm, buf):
        wid = lax.axis_index("s") * _NC + lax.axis_index("c")
        base = wid * _ROWS_PER_W
        for c in range(_ROWS_PER_W // _CHUNK):
            start = base + c * _CHUNK
            pltpu.sync_copy(table_hbm.at[pl.ds(start, _CHUNK)], buf)
            for b in range(_B):
                pltpu.sync_copy(buf, out_hbm.at[b, pl.ds(start, _CHUNK)])

    return pe_kernel


_pe = _make_pe_kernel()


def kernel(seq_h, pos_table):
    del seq_h
    return _pe(pos_table)
